# baseline (speedup 1.0000x reference)
import functools

import jax
import jax.numpy as jnp
from jax import lax
from jax.experimental import pallas as pl
from jax.experimental.pallas import tpu as pltpu
from jax.experimental.pallas import tpu_sc as plsc

D_MODEL = 768
RANK = 64
N_EXPERT = 64
S_TILE = 256
NG = 8
GROUPS = N_EXPERT // NG
W32 = RANK // 2


def _relayout_body(w_hbm, out_hbm, buf):
    wid = lax.axis_index("s") * 2 + lax.axis_index("c")
    for k in range(2):
        n = wid * 2 + k
        pltpu.sync_copy(w_hbm.at[n], buf)
        pltpu.sync_copy(buf, out_hbm.at[:, pl.ds(n * W32, W32)])


def _sc_relayout(w_i32):
    mesh = plsc.VectorSubcoreMesh(core_axis_name="c", subcore_axis_name="s")
    kern = functools.partial(
        pl.kernel,
        out_type=jax.ShapeDtypeStruct((D_MODEL, N_EXPERT * W32), jnp.int32),
        mesh=mesh,
        scratch_types=[pltpu.VMEM((D_MODEL, W32), jnp.int32)],
        compiler_params=pltpu.CompilerParams(use_tc_tiling_on_sc=False),
    )(_relayout_body)
    return kern(w_i32)


def _main_body(x_ref, rwt_ref, wflat_ref, expand_ref, reduce_ref,
               out_ref, idx_ref, w_out_ref):
    x = x_ref[...]

    scores = jax.lax.dot_general(
        x, rwt_ref[...], (((1,), (0,)), ((), ())),
        preferred_element_type=jnp.float32)

    iota = jax.lax.broadcasted_iota(jnp.int32, (S_TILE, N_EXPERT), 1)
    m1 = jnp.max(scores, axis=1, keepdims=True)
    i1 = jnp.min(jnp.where(scores == m1, iota, N_EXPERT), axis=1,
                 keepdims=True)
    masked = jnp.where(iota == i1, -jnp.inf, scores)
    m2 = jnp.max(masked, axis=1, keepdims=True)
    i2 = jnp.min(jnp.where(masked == m2, iota, N_EXPERT), axis=1,
                 keepdims=True)

    e = jnp.exp(m2 - m1)
    denom = 1.0 + e
    w1 = 1.0 / denom
    w2 = e / denom

    idx_ref[...] = jnp.concatenate([i1, i2], axis=1)
    w_out_ref[...] = jnp.concatenate([w1, w2], axis=1)

    comb = jnp.where(iota == i1, w1, 0.0) + jnp.where(iota == i2, w2, 0.0)
    comb_bf = comb.astype(jnp.bfloat16)

    x_bf = x.astype(jnp.bfloat16)
    acc = jnp.zeros((S_TILE, RANK), dtype=jnp.float32)
    for g in range(GROUPS):
        sl = slice(NG * RANK * g, NG * RANK * (g + 1))
        combfull = jax.lax.dot_general(
            comb_bf, expand_ref[:, sl], (((1,), (0,)), ((), ())),
            preferred_element_type=jnp.float32).astype(jnp.bfloat16)
        proj = jax.lax.dot_general(
            x_bf, wflat_ref[:, sl], (((1,), (0,)), ((), ())),
            preferred_element_type=jnp.float32)
        cp = proj.astype(jnp.bfloat16) * combfull
        acc = acc + jax.lax.dot_general(
            cp, reduce_ref[sl, :], (((1,), (0,)), ((), ())),
            preferred_element_type=jnp.float32)
    out_ref[...] = acc


@jax.jit
def kernel(x, router_w, compress_neurons):
    b, s, d = x.shape
    xs = x.reshape(s, d)
    rwt = router_w.T

    cols = N_EXPERT * RANK
    w_bf = compress_neurons.astype(jnp.bfloat16)
    w_i32 = jax.lax.bitcast_convert_type(
        w_bf.reshape(N_EXPERT, d, W32, 2), jnp.int32)
    wflat_i32 = _sc_relayout(w_i32)
    wflat = jax.lax.bitcast_convert_type(
        wflat_i32, jnp.bfloat16).reshape(d, cols)

    c_iota = jnp.arange(cols, dtype=jnp.int32)
    expand = (jnp.arange(N_EXPERT, dtype=jnp.int32)[:, None]
              == (c_iota[None, :] // RANK)).astype(jnp.bfloat16)
    reduce = ((c_iota[:, None] % RANK)
              == jnp.arange(RANK, dtype=jnp.int32)[None, :]
              ).astype(jnp.bfloat16)

    grid = (s // S_TILE,)
    out, idx, w = pl.pallas_call(
        _main_body,
        grid=grid,
        in_specs=[
            pl.BlockSpec((S_TILE, d), lambda i: (i, 0)),
            pl.BlockSpec((d, N_EXPERT), lambda i: (0, 0)),
            pl.BlockSpec((d, cols), lambda i: (0, 0)),
            pl.BlockSpec((N_EXPERT, cols), lambda i: (0, 0)),
            pl.BlockSpec((cols, RANK), lambda i: (0, 0)),
        ],
        out_specs=[
            pl.BlockSpec((S_TILE, RANK), lambda i: (i, 0)),
            pl.BlockSpec((S_TILE, 2), lambda i: (i, 0)),
            pl.BlockSpec((S_TILE, 2), lambda i: (i, 0)),
        ],
        out_shape=[
            jax.ShapeDtypeStruct((s, RANK), jnp.float32),
            jax.ShapeDtypeStruct((s, 2), jnp.int32),
            jax.ShapeDtypeStruct((s, 2), jnp.float32),
        ],
    )(xs, rwt, wflat, expand, reduce)
    return (out.reshape(b, s, RANK), idx.reshape(b, s, 2),
            w.reshape(b, s, 2))


# speedup vs baseline: 3.0651x; 3.0651x over previous
import jax
import jax.numpy as jnp
from jax.experimental import pallas as pl
from jax.experimental.pallas import tpu as pltpu

D_MODEL = 768
RANK = 64
N_EXPERT = 64
S_TILE = 256
NG = 8
GROUPS = N_EXPERT // NG


def _main_body(*refs):
    x_ref, rwt_ref, expand_ref, reduce_ref = refs[:4]
    w_refs = refs[4:4 + N_EXPERT]
    out_ref, idx_ref, w_out_ref = refs[4 + N_EXPERT:]

    x = x_ref[...]

    scores = jax.lax.dot_general(
        x, rwt_ref[...], (((1,), (0,)), ((), ())),
        preferred_element_type=jnp.float32)

    iota = jax.lax.broadcasted_iota(jnp.int32, (S_TILE, N_EXPERT), 1)
    m1 = jnp.max(scores, axis=1, keepdims=True)
    i1 = jnp.min(jnp.where(scores == m1, iota, N_EXPERT), axis=1,
                 keepdims=True)
    masked = jnp.where(iota == i1, -jnp.inf, scores)
    m2 = jnp.max(masked, axis=1, keepdims=True)
    i2 = jnp.min(jnp.where(masked == m2, iota, N_EXPERT), axis=1,
                 keepdims=True)

    e = jnp.exp(m2 - m1)
    denom = 1.0 + e
    w1 = 1.0 / denom
    w2 = e / denom

    idx_ref[...] = jnp.concatenate([i1, i2], axis=1)
    w_out_ref[...] = jnp.concatenate([w1, w2], axis=1)

    comb = jnp.where(iota == i1, w1, 0.0) + jnp.where(iota == i2, w2, 0.0)
    comb_bf = comb.astype(jnp.bfloat16)

    x_bf = x.astype(jnp.bfloat16)
    acc = jnp.zeros((S_TILE, RANK), dtype=jnp.float32)
    for g in range(GROUPS):
        sl = slice(NG * RANK * g, NG * RANK * (g + 1))
        wbig = jnp.concatenate(
            [w_refs[NG * g + j][...] for j in range(NG)], axis=1)
        combfull = jax.lax.dot_general(
            comb_bf, expand_ref[:, sl], (((1,), (0,)), ((), ())),
            preferred_element_type=jnp.float32).astype(jnp.bfloat16)
        proj = jax.lax.dot_general(
            x_bf, wbig, (((1,), (0,)), ((), ())),
            preferred_element_type=jnp.float32)
        cp = proj.astype(jnp.bfloat16) * combfull
        acc = acc + jax.lax.dot_general(
            cp, reduce_ref[sl, :], (((1,), (0,)), ((), ())),
            preferred_element_type=jnp.float32)
    out_ref[...] = acc


@jax.jit
def kernel(x, router_w, compress_neurons):
    b, s, d = x.shape
    xs = x.reshape(s, d)
    rwt = router_w.T
    w_bf = compress_neurons.astype(jnp.bfloat16)
    w_blocks = [w_bf[n] for n in range(N_EXPERT)]

    cols = N_EXPERT * RANK
    c_iota = jnp.arange(cols, dtype=jnp.int32)
    expand = (jnp.arange(N_EXPERT, dtype=jnp.int32)[:, None]
              == (c_iota[None, :] // RANK)).astype(jnp.bfloat16)
    reduce = ((c_iota[:, None] % RANK)
              == jnp.arange(RANK, dtype=jnp.int32)[None, :]
              ).astype(jnp.bfloat16)

    grid = (s // S_TILE,)
    out, idx, w = pl.pallas_call(
        _main_body,
        grid=grid,
        in_specs=[
            pl.BlockSpec((S_TILE, d), lambda i: (i, 0)),
            pl.BlockSpec((d, N_EXPERT), lambda i: (0, 0)),
            pl.BlockSpec((N_EXPERT, cols), lambda i: (0, 0)),
            pl.BlockSpec((cols, RANK), lambda i: (0, 0)),
        ] + [pl.BlockSpec((d, RANK), lambda i: (0, 0))] * N_EXPERT,
        out_specs=[
            pl.BlockSpec((S_TILE, RANK), lambda i: (i, 0)),
            pl.BlockSpec((S_TILE, 2), lambda i: (i, 0)),
            pl.BlockSpec((S_TILE, 2), lambda i: (i, 0)),
        ],
        out_shape=[
            jax.ShapeDtypeStruct((s, RANK), jnp.float32),
            jax.ShapeDtypeStruct((s, 2), jnp.int32),
            jax.ShapeDtypeStruct((s, 2), jnp.float32),
        ],
    )(xs, rwt, expand, reduce, *w_blocks)
    return (out.reshape(b, s, RANK), idx.reshape(b, s, 2),
            w.reshape(b, s, 2))


# speedup vs baseline: 4.1589x; 1.3569x over previous
import jax
import jax.numpy as jnp
from jax.experimental import pallas as pl

D_MODEL = 768
RANK = 64
N_EXPERT = 64
S_TILE = 512
NG = 8
GROUPS = N_EXPERT // NG


def _main_body(x_ref, rwt_ref, wflat_ref, expand_ref, reduce_ref,
               out_ref, idx_ref, w_out_ref):
    x = x_ref[...]

    scores = jax.lax.dot_general(
        x, rwt_ref[...], (((1,), (0,)), ((), ())),
        preferred_element_type=jnp.float32)

    iota = jax.lax.broadcasted_iota(jnp.int32, (S_TILE, N_EXPERT), 1)
    m1 = jnp.max(scores, axis=1, keepdims=True)
    i1 = jnp.min(jnp.where(scores == m1, iota, N_EXPERT), axis=1,
                 keepdims=True)
    masked = jnp.where(iota == i1, -jnp.inf, scores)
    m2 = jnp.max(masked, axis=1, keepdims=True)
    i2 = jnp.min(jnp.where(masked == m2, iota, N_EXPERT), axis=1,
                 keepdims=True)

    e = jnp.exp(m2 - m1)
    denom = 1.0 + e
    w1 = 1.0 / denom
    w2 = e / denom

    idx_ref[...] = jnp.concatenate([i1, i2], axis=1)
    w_out_ref[...] = jnp.concatenate([w1, w2], axis=1)

    comb = jnp.where(iota == i1, w1, 0.0) + jnp.where(iota == i2, w2, 0.0)
    comb_bf = comb.astype(jnp.bfloat16)

    x_bf = x.astype(jnp.bfloat16)
    acc = jnp.zeros((S_TILE, RANK), dtype=jnp.float32)
    for g in range(GROUPS):
        sl = slice(NG * RANK * g, NG * RANK * (g + 1))
        combfull = jax.lax.dot_general(
            comb_bf, expand_ref[:, sl], (((1,), (0,)), ((), ())),
            preferred_element_type=jnp.float32).astype(jnp.bfloat16)
        proj = jax.lax.dot_general(
            x_bf, wflat_ref[:, sl], (((1,), (0,)), ((), ())),
            preferred_element_type=jnp.float32)
        cp = proj.astype(jnp.bfloat16) * combfull
        acc = acc + jax.lax.dot_general(
            cp, reduce_ref[sl, :], (((1,), (0,)), ((), ())),
            preferred_element_type=jnp.float32)
    out_ref[...] = acc


@jax.jit
def kernel(x, router_w, compress_neurons):
    b, s, d = x.shape
    xs = x.reshape(s, d)
    rwt = router_w.T

    cols = N_EXPERT * RANK
    wflat = compress_neurons.astype(jnp.bfloat16).transpose(1, 0, 2)
    wflat = wflat.reshape(d, cols)

    c_iota = jnp.arange(cols, dtype=jnp.int32)
    expand = (jnp.arange(N_EXPERT, dtype=jnp.int32)[:, None]
              == (c_iota[None, :] // RANK)).astype(jnp.bfloat16)
    reduce = ((c_iota[:, None] % RANK)
              == jnp.arange(RANK, dtype=jnp.int32)[None, :]
              ).astype(jnp.bfloat16)

    grid = (s // S_TILE,)
    out, idx, w = pl.pallas_call(
        _main_body,
        grid=grid,
        in_specs=[
            pl.BlockSpec((S_TILE, d), lambda i: (i, 0)),
            pl.BlockSpec((d, N_EXPERT), lambda i: (0, 0)),
            pl.BlockSpec((d, cols), lambda i: (0, 0)),
            pl.BlockSpec((N_EXPERT, cols), lambda i: (0, 0)),
            pl.BlockSpec((cols, RANK), lambda i: (0, 0)),
        ],
        out_specs=[
            pl.BlockSpec((S_TILE, RANK), lambda i: (i, 0)),
            pl.BlockSpec((S_TILE, 2), lambda i: (i, 0)),
            pl.BlockSpec((S_TILE, 2), lambda i: (i, 0)),
        ],
        out_shape=[
            jax.ShapeDtypeStruct((s, RANK), jnp.float32),
            jax.ShapeDtypeStruct((s, 2), jnp.int32),
            jax.ShapeDtypeStruct((s, 2), jnp.float32),
        ],
    )(xs, rwt, wflat, expand, reduce)
    return (out.reshape(b, s, RANK), idx.reshape(b, s, 2),
            w.reshape(b, s, 2))


# speedup vs baseline: 4.2747x; 1.0278x over previous
import jax
import jax.numpy as jnp
from jax.experimental import pallas as pl

D_MODEL = 768
RANK = 64
N_EXPERT = 64
S_TILE = 1024
NG = 8
GROUPS = N_EXPERT // NG


def _main_body(x_ref, rwt_ref, wflat_ref, expand_ref, reduce_ref,
               out_ref, idx_ref, w_out_ref):
    x = x_ref[...]

    scores = jax.lax.dot_general(
        x, rwt_ref[...], (((1,), (0,)), ((), ())),
        preferred_element_type=jnp.float32)

    iota = jax.lax.broadcasted_iota(jnp.int32, (S_TILE, N_EXPERT), 1)
    m1 = jnp.max(scores, axis=1, keepdims=True)
    i1 = jnp.min(jnp.where(scores == m1, iota, N_EXPERT), axis=1,
                 keepdims=True)
    masked = jnp.where(iota == i1, -jnp.inf, scores)
    m2 = jnp.max(masked, axis=1, keepdims=True)
    i2 = jnp.min(jnp.where(masked == m2, iota, N_EXPERT), axis=1,
                 keepdims=True)

    e = jnp.exp(m2 - m1)
    denom = 1.0 + e
    w1 = 1.0 / denom
    w2 = e / denom

    idx_ref[...] = jnp.concatenate([i1, i2], axis=1)
    w_out_ref[...] = jnp.concatenate([w1, w2], axis=1)

    comb = jnp.where(iota == i1, w1, 0.0) + jnp.where(iota == i2, w2, 0.0)
    comb_bf = comb.astype(jnp.bfloat16)

    x_bf = x.astype(jnp.bfloat16)
    acc = jnp.zeros((S_TILE, RANK), dtype=jnp.float32)
    for g in range(GROUPS):
        sl = slice(NG * RANK * g, NG * RANK * (g + 1))
        combfull = jax.lax.dot_general(
            comb_bf, expand_ref[:, sl], (((1,), (0,)), ((), ())),
            preferred_element_type=jnp.float32).astype(jnp.bfloat16)
        proj = jax.lax.dot_general(
            x_bf, wflat_ref[:, sl], (((1,), (0,)), ((), ())),
            preferred_element_type=jnp.float32)
        cp = proj.astype(jnp.bfloat16) * combfull
        acc = acc + jax.lax.dot_general(
            cp, reduce_ref[sl, :], (((1,), (0,)), ((), ())),
            preferred_element_type=jnp.float32)
    out_ref[...] = acc


@jax.jit
def kernel(x, router_w, compress_neurons):
    b, s, d = x.shape
    xs = x.reshape(s, d)
    rwt = router_w.T

    cols = N_EXPERT * RANK
    wflat = compress_neurons.astype(jnp.bfloat16).transpose(1, 0, 2)
    wflat = wflat.reshape(d, cols)

    c_iota = jnp.arange(cols, dtype=jnp.int32)
    expand = (jnp.arange(N_EXPERT, dtype=jnp.int32)[:, None]
              == (c_iota[None, :] // RANK)).astype(jnp.bfloat16)
    reduce = ((c_iota[:, None] % RANK)
              == jnp.arange(RANK, dtype=jnp.int32)[None, :]
              ).astype(jnp.bfloat16)

    grid = (s // S_TILE,)
    out, idx, w = pl.pallas_call(
        _main_body,
        grid=grid,
        in_specs=[
            pl.BlockSpec((S_TILE, d), lambda i: (i, 0)),
            pl.BlockSpec((d, N_EXPERT), lambda i: (0, 0)),
            pl.BlockSpec((d, cols), lambda i: (0, 0)),
            pl.BlockSpec((N_EXPERT, cols), lambda i: (0, 0)),
            pl.BlockSpec((cols, RANK), lambda i: (0, 0)),
        ],
        out_specs=[
            pl.BlockSpec((S_TILE, RANK), lambda i: (i, 0)),
            pl.BlockSpec((S_TILE, 2), lambda i: (i, 0)),
            pl.BlockSpec((S_TILE, 2), lambda i: (i, 0)),
        ],
        out_shape=[
            jax.ShapeDtypeStruct((s, RANK), jnp.float32),
            jax.ShapeDtypeStruct((s, 2), jnp.int32),
            jax.ShapeDtypeStruct((s, 2), jnp.float32),
        ],
    )(xs, rwt, wflat, expand, reduce)
    return (out.reshape(b, s, RANK), idx.reshape(b, s, 2),
            w.reshape(b, s, 2))
